# baseline (speedup 1.0000x reference)
import jax
import jax.numpy as jnp
from jax.experimental import pallas as pl

_B = 1024
_M = 8
_K = 32
_D = 128
_OUT = 128
_TR = 512
_H = _K // 2


def _body(sv_ref, nba_ref, nbb_ref, nbc_ref, nbd_ref, mk_ref, w_ref, b_ref,
          out_ref):
    m = mk_ref[...]
    w = w_ref[...]
    sv = sv_ref[...]

    scale = 1.0 / _K
    e0 = (jnp.sum(nba_ref[:, 0] * m[:, 0 * _H:1 * _H, None], axis=1)
          + jnp.sum(nbb_ref[:, 0] * m[:, 1 * _H:2 * _H, None], axis=1)) * scale
    e1 = (jnp.sum(nbc_ref[:, 0] * m[:, 2 * _H:3 * _H, None], axis=1)
          + jnp.sum(nbd_ref[:, 0] * m[:, 3 * _H:4 * _H, None], axis=1)) * scale

    dn = (((1,), (1,)), ((), ()))
    acc = jax.lax.dot_general(sv, w[:, :_D], dn,
                              preferred_element_type=jnp.float32)
    acc += jax.lax.dot_general(e0, w[:, _D:2 * _D], dn,
                               preferred_element_type=jnp.float32)
    acc += jax.lax.dot_general(e1, w[:, 2 * _D:], dn,
                               preferred_element_type=jnp.float32)
    out_ref[...] = acc + b_ref[...]


def kernel(self_vectors, neighbor_vectors, masks, W, b):
    R = _B * _M
    nb = neighbor_vectors.reshape(R, 4, _H, _D)
    mk = masks.reshape(R, 2 * _K)
    sv = self_vectors.reshape(R, _D)
    b2 = b.reshape(1, _OUT)

    def nbspec(j):
        return pl.BlockSpec((_TR, 1, _H, _D), lambda i, j=j: (i, j, 0, 0))

    grid = (R // _TR,)
    out = pl.pallas_call(
        _body,
        grid=grid,
        in_specs=[
            pl.BlockSpec((_TR, _D), lambda i: (i, 0)),
            nbspec(0), nbspec(1), nbspec(2), nbspec(3),
            pl.BlockSpec((_TR, 2 * _K), lambda i: (i, 0)),
            pl.BlockSpec((_OUT, 3 * _D), lambda i: (0, 0)),
            pl.BlockSpec((1, _OUT), lambda i: (0, 0)),
        ],
        out_specs=pl.BlockSpec((_TR, _OUT), lambda i: (i, 0)),
        out_shape=jax.ShapeDtypeStruct((R, _OUT), jnp.float32),
    )(sv, nb, nb, nb, nb, mk, W, b2)
    return out.reshape(_B, _M, _OUT)
